# baseline (speedup 1.0000x reference)
import functools

import jax
import jax.numpy as jnp
from jax import lax
from jax.experimental import pallas as pl
from jax.experimental.pallas import tpu as pltpu
from jax.experimental.pallas import tpu_sc as plsc

B = 16
SL = 9
VIN = [16384, 4096, 1024, 256]
VOUT = [4096, 1024, 256, 64]
CH = [3, 64, 128, 256, 512]
LATENT = 256

NC = 2
NS = 16
NW = NC * NS


def _mesh():
    return plsc.VectorSubcoreMesh(
        core_axis_name="c", subcore_axis_name="s", num_cores=NC, num_subcores=NS
    )


def _wid():
    return lax.axis_index("s") * NC + lax.axis_index("c")


def _chunk(n):
    for c in range(min(n, 128), 0, -1):
        if n % c == 0 and (c % 8 == 0 or c == n):
            return c
    return n


_SC_PARAMS = dict(
    compiler_params=pltpu.CompilerParams(use_tc_tiling_on_sc=False),
)


def _make_compose(V, N):
    jn = N // NW
    CG = _chunk(jn)
    nch = jn // CG

    @functools.partial(
        pl.kernel,
        mesh=_mesh(),
        out_type=jax.ShapeDtypeStruct((N, 16), jnp.int32),
        scratch_types=[
            pltpu.VMEM((nch, CG), jnp.int32),
            pltpu.VMEM((jn, 16), jnp.int32),
            pltpu.SemaphoreType.DMA,
        ],
        **_SC_PARAMS,
    )
    def k(sp_hbm, dc_hbm, out_hbm, dc_v, rows_v, sem):
        base = _wid() * jn
        pltpu.sync_copy(dc_hbm.at[_wid()], dc_v)

        def issue(ci, carry):
            pltpu.async_copy(
                sp_hbm.at[dc_v.at[ci]], rows_v.at[pl.ds(ci * CG, CG)], sem
            )
            return carry

        lax.fori_loop(0, nch, issue, 0)
        pltpu.make_async_copy(
            out_hbm.at[pl.ds(base, jn)], rows_v, sem
        ).wait()
        pltpu.sync_copy(rows_v, out_hbm.at[pl.ds(base, jn)])

    return k


def _make_gather(V, C, N):
    jw = N // NW
    CG = _chunk(jw)
    nch = jw // CG

    @functools.partial(
        pl.kernel,
        mesh=_mesh(),
        out_type=jax.ShapeDtypeStruct((SL, B, N, C), jnp.float32),
        scratch_types=[
            pltpu.VMEM((nch, CG), jnp.int32),
            pltpu.VMEM((jw, C), jnp.float32),
            pltpu.SemaphoreType.DMA,
        ],
        **_SC_PARAMS,
    )
    def k(x_hbm, idxT_hbm, g_hbm, idx_v, rows_v, sem):
        wid = _wid()
        base = wid * jw

        def s_body(s, c0):
            pltpu.sync_copy(idxT_hbm.at[s].at[wid], idx_v)

            def b_body(b, c1):
                def issue(ci, c2):
                    pltpu.async_copy(
                        x_hbm.at[b].at[idx_v.at[ci]],
                        rows_v.at[pl.ds(ci * CG, CG)],
                        sem,
                    )
                    return c2

                lax.fori_loop(0, nch, issue, 0)
                dst = g_hbm.at[s].at[b].at[pl.ds(base, jw)]
                pltpu.make_async_copy(dst, rows_v, sem).wait()
                pltpu.sync_copy(rows_v, dst)
                return c1

            lax.fori_loop(0, B, b_body, 0)
            return c0

        lax.fori_loop(0, SL, s_body, 0)

    return k


def _make_scatter(U, D, N):
    jt = N // NS
    bpb = U * D
    Bg = min(B // NC, (6 * 2 ** 20 // 4) // bpb)
    while (B // NC) % Bg:
        Bg -= 1
    passes = (B // NC) // Bg
    rpt = U // NS
    jc = _chunk(jt)
    nchs = jt // jc
    LY = min(jt, 512)
    nld = jt // LY
    spc = LY // jc

    @functools.partial(
        pl.kernel,
        mesh=_mesh(),
        out_type=jax.ShapeDtypeStruct((B, U, D), jnp.float32),
        scratch_types=[
            pltpu.VMEM_SHARED((Bg * U, D), jnp.float32),
            pltpu.VMEM((LY, D), jnp.float32),
            pltpu.VMEM((jt,), jnp.int32),
            pltpu.VMEM((nchs, jc), jnp.int32),
            pltpu.SemaphoreType.DMA,
        ],
        **_SC_PARAMS,
    )
    def k(y_hbm, dr_hbm, z_hbm, out_hbm, acc_sh, y_v, idx_v, idxb_v, sem):
        cid = lax.axis_index("c")
        sid = lax.axis_index("s")
        jbase = sid * jt
        pltpu.sync_copy(dr_hbm.at[sid], idx_v)
        for p in range(passes):
            pltpu.sync_copy(z_hbm, acc_sh.at[pl.ds(sid * Bg * rpt, Bg * rpt)])
            plsc.subcore_barrier()

            def batch_body(bl, carry):
                b = cid * (B // NC) + p * Bg + bl

                def off_body(kk, c2):
                    r_ = kk // (jc // 16)
                    k16 = kk % (jc // 16)
                    idxb_v[r_, pl.ds(k16 * 16, 16)] = (
                        idx_v[pl.ds(kk * 16, 16)] + bl * U
                    )
                    return c2

                lax.fori_loop(0, jt // 16, off_body, 0)

                def load_body(q, c2):
                    pltpu.sync_copy(
                        y_hbm.at[b].at[pl.ds(jbase + q * LY, LY)], y_v
                    )

                    def scat_body(r0, c3):
                        r_ = q * spc + r0
                        pltpu.sync_copy(
                            y_v.at[pl.ds(r0 * jc, jc)],
                            acc_sh.at[idxb_v.at[r_]],
                            add=True,
                        )
                        return c3

                    lax.fori_loop(0, spc, scat_body, 0)
                    return c2

                lax.fori_loop(0, nld, load_body, 0)
                return carry

            lax.fori_loop(0, Bg, batch_body, 0)
            plsc.subcore_barrier()

            def out_body(bl, carry):
                b = cid * (B // NC) + p * Bg + bl
                pltpu.sync_copy(
                    acc_sh.at[pl.ds(bl * U + sid * rpt, rpt)],
                    out_hbm.at[b].at[pl.ds(sid * rpt, rpt)],
                )
                return carry

            lax.fori_loop(0, Bg, out_body, 0)
            if p + 1 < passes:
                plsc.subcore_barrier()

    return k


def _matmul_elu_scale(g, w3, bias, dv2):
    _, Bb, N, C = g.shape
    D = w3.shape[2]
    BN = min(N, 1024)

    def body(g_ref, w_ref, b_ref, s_ref, o_ref):
        acc = jnp.broadcast_to(b_ref[...], (BN, D))
        for s in range(SL):
            acc = acc + jnp.dot(
                g_ref[s, 0], w_ref[s], preferred_element_type=jnp.float32
            )
        acc = jnp.where(acc > 0, acc, jnp.exp(jnp.minimum(acc, 0.0)) - 1.0)
        o_ref[0] = acc * s_ref[...]

    return pl.pallas_call(
        body,
        grid=(Bb, N // BN),
        in_specs=[
            pl.BlockSpec((SL, 1, BN, C), lambda b, r: (0, b, r, 0)),
            pl.BlockSpec((SL, C, D), lambda b, r: (0, 0, 0)),
            pl.BlockSpec((1, D), lambda b, r: (0, 0)),
            pl.BlockSpec((BN, 1), lambda b, r: (r, 0)),
        ],
        out_specs=pl.BlockSpec((1, BN, D), lambda b, r: (b, r, 0)),
        out_shape=jax.ShapeDtypeStruct((Bb, N, D), jnp.float32),
    )(g, w3, bias, dv2)


def _final_matmul(h, wf3, bf):
    Bb, U, D = h.shape
    L = wf3.shape[2]
    VB = 8

    def body(h_ref, w_ref, b_ref, o_ref):
        @pl.when(pl.program_id(0) == 0)
        def _():
            o_ref[...] = jnp.broadcast_to(b_ref[...], o_ref.shape)

        acc = jnp.zeros((Bb, L), jnp.float32)
        for v in range(VB):
            acc = acc + jnp.dot(
                h_ref[:, v, :], w_ref[v], preferred_element_type=jnp.float32
            )
        o_ref[...] += acc

    return pl.pallas_call(
        body,
        grid=(U // VB,),
        in_specs=[
            pl.BlockSpec((Bb, VB, D), lambda k: (0, k, 0)),
            pl.BlockSpec((VB, D, L), lambda k: (k, 0, 0)),
            pl.BlockSpec((1, L), lambda k: (0, 0)),
        ],
        out_specs=pl.BlockSpec((Bb, L), lambda k: (0, 0)),
        out_shape=jax.ShapeDtypeStruct((Bb, L), jnp.float32),
    )(h, wf3, bf)


def kernel(x, W0, b0, W1, b1, W2, b2, W3, b3, Wf, bf,
           dv0, dv1, dv2, dv3, sp0, sp1, sp2, sp3,
           dr0, dr1, dr2, dr3, dc0, dc1, dc2, dc3):
    Ws = [W0, W1, W2, W3]
    bs = [b0, b1, b2, b3]
    sps = [sp0, sp1, sp2, sp3]
    drs = [dr0, dr1, dr2, dr3]
    dcs = [dc0, dc1, dc2, dc3]
    dvs = [dv0, dv1, dv2, dv3]

    h = jnp.pad(x, ((0, 0), (0, 0), (0, 13)))
    W0p = jnp.zeros((SL, 16, CH[1]), W0.dtype).at[:, :3, :].set(
        W0.reshape(SL, 3, CH[1])
    )
    Ws3 = [W0p, W1.reshape(SL, CH[1], CH[2]), W2.reshape(SL, CH[2], CH[3]),
           W3.reshape(SL, CH[3], CH[4])]
    Cg = [16, 64, 128, 256]

    for i in range(4):
        V, U, C, D = VIN[i], VOUT[i], Cg[i], CH[i + 1]
        N = 4 * U
        jn = N // NW
        dc3 = dcs[i].reshape(NW, jn // _chunk(jn), _chunk(jn))
        idx2 = _make_compose(V, N)(
            jnp.pad(sps[i], ((0, 0), (0, 16 - SL))), dc3
        )
        jw = N // NW
        cg = _chunk(jw)
        idxT = idx2[:, :SL].transpose(1, 0).reshape(SL, NW, jw // cg, cg)
        g = _make_gather(V, C, N)(h, idxT)
        y = _matmul_elu_scale(
            g, Ws3[i], bs[i].reshape(1, D), dvs[i].reshape(N, 1)
        )
        Bg = min(B // NC, (6 * 2 ** 20 // 4) // (U * D))
        while (B // NC) % Bg:
            Bg -= 1
        z = jnp.zeros((Bg * (U // NS), D), jnp.float32)
        h = _make_scatter(U, D, N)(y, drs[i].reshape(NS, N // NS), z)
    z = _final_matmul(
        h, Wf.reshape(VOUT[3], CH[4], LATENT), bf.reshape(1, LATENT)
    )
    return z


# speedup vs baseline: 1.5386x; 1.5386x over previous
import functools

import jax
import jax.numpy as jnp
from jax import lax
from jax.experimental import pallas as pl
from jax.experimental.pallas import tpu as pltpu
from jax.experimental.pallas import tpu_sc as plsc

B = 16
SL = 9
VIN = [16384, 4096, 1024, 256]
VOUT = [4096, 1024, 256, 64]
CH = [3, 64, 128, 256, 512]
LATENT = 256

NC = 2
NS = 16
NW = NC * NS


def _mesh():
    return plsc.VectorSubcoreMesh(
        core_axis_name="c", subcore_axis_name="s", num_cores=NC, num_subcores=NS
    )


def _wid():
    return lax.axis_index("s") * NC + lax.axis_index("c")


def _chunk(n):
    for c in range(min(n, 128), 0, -1):
        if n % c == 0 and (c % 8 == 0 or c == n):
            return c
    return n


_SC_PARAMS = dict(
    compiler_params=pltpu.CompilerParams(use_tc_tiling_on_sc=False),
)


def _make_compose(V, N):
    jn = N // NW
    CG = _chunk(jn)
    nch = jn // CG

    @functools.partial(
        pl.kernel,
        mesh=_mesh(),
        out_type=jax.ShapeDtypeStruct((N, 16), jnp.int32),
        scratch_types=[
            pltpu.VMEM((nch, CG), jnp.int32),
            pltpu.VMEM((jn, 16), jnp.int32),
            pltpu.SemaphoreType.DMA,
        ],
        **_SC_PARAMS,
    )
    def k(sp_hbm, dc_hbm, out_hbm, dc_v, rows_v, sem):
        base = _wid() * jn
        pltpu.sync_copy(dc_hbm.at[_wid()], dc_v)

        def issue(ci, carry):
            pltpu.async_copy(
                sp_hbm.at[dc_v.at[ci]], rows_v.at[pl.ds(ci * CG, CG)], sem
            )
            return carry

        lax.fori_loop(0, nch, issue, 0)
        pltpu.make_async_copy(
            out_hbm.at[pl.ds(base, jn)], rows_v, sem
        ).wait()
        pltpu.sync_copy(rows_v, out_hbm.at[pl.ds(base, jn)])

    return k


def _make_gather(V, C, N):
    R = (N * SL) // NW
    CG = _chunk(R)
    nch = R // CG
    nbuf = 2 if 2 * R * C * 4 + nch * CG * 4 <= 400_000 else 1

    @functools.partial(
        pl.kernel,
        mesh=_mesh(),
        out_type=jax.ShapeDtypeStruct((B * N * SL, C), jnp.float32),
        scratch_types=[
            pltpu.VMEM((nch, CG), jnp.int32),
            pltpu.VMEM((nbuf, R, C), jnp.float32),
            pltpu.SemaphoreType.DMA,
            pltpu.SemaphoreType.DMA,
            pltpu.SemaphoreType.DMA,
            pltpu.SemaphoreType.DMA,
        ],
        **_SC_PARAMS,
    )
    def k(x_hbm, idx_hbm, g_hbm, idx_v, rows_v, g0, g1, t0, t1):
        wid = _wid()
        base = wid * R
        pltpu.sync_copy(idx_hbm.at[wid], idx_v)
        gsems = [g0, g1]
        ssems = [t0, t1]

        def gdst(b):
            return g_hbm.at[pl.ds(b * N * SL + base, R)]

        def fire(b, buf):
            def issue(ci, c2):
                pltpu.async_copy(
                    x_hbm.at[b].at[idx_v.at[ci]],
                    rows_v.at[buf].at[pl.ds(ci * CG, CG)],
                    gsems[buf],
                )
                return c2

            lax.fori_loop(0, nch, issue, 0)

        def drain_then_store(b, buf):
            pltpu.make_async_copy(gdst(b), rows_v.at[buf], gsems[buf]).wait()
            pltpu.async_copy(rows_v.at[buf], gdst(b), ssems[buf])

        def wait_store(b, buf):
            pltpu.make_async_copy(rows_v.at[buf], gdst(b), ssems[buf]).wait()

        if nbuf == 2:
            fire(0, 0)
            for b in range(1, B):
                buf = b % 2
                if b >= 2:
                    wait_store(b - 2, buf)
                fire(b, buf)
                drain_then_store(b - 1, 1 - buf)
            drain_then_store(B - 1, (B - 1) % 2)
            wait_store(B - 2, B % 2)
            wait_store(B - 1, (B - 1) % 2)
        else:
            def b_body(b, c1):
                fire(b, 0)
                pltpu.make_async_copy(gdst(b), rows_v.at[0], gsems[0]).wait()
                pltpu.sync_copy(rows_v.at[0], gdst(b))
                return c1

            lax.fori_loop(0, B, b_body, 0)

    return k


def _make_scatter(U, D, N):
    jt = N // NS
    bpb = U * D
    Bg = min(B // NC, (6 * 2 ** 20 // 4) // bpb)
    while (B // NC) % Bg:
        Bg -= 1
    passes = (B // NC) // Bg
    rpt = U // NS
    jc = _chunk(jt)
    nchs = jt // jc
    LY = min(jt, 512)
    nld = jt // LY
    spc = LY // jc

    @functools.partial(
        pl.kernel,
        mesh=_mesh(),
        out_type=jax.ShapeDtypeStruct((B, U, D), jnp.float32),
        scratch_types=[
            pltpu.VMEM_SHARED((Bg * U, D), jnp.float32),
            pltpu.VMEM((LY, D), jnp.float32),
            pltpu.VMEM((jt,), jnp.int32),
            pltpu.VMEM((nchs, jc), jnp.int32),
            pltpu.SemaphoreType.DMA,
        ],
        **_SC_PARAMS,
    )
    def k(y_hbm, dr_hbm, z_hbm, out_hbm, acc_sh, y_v, idx_v, idxb_v, sem):
        cid = lax.axis_index("c")
        sid = lax.axis_index("s")
        jbase = sid * jt
        pltpu.sync_copy(dr_hbm.at[sid], idx_v)
        for p in range(passes):
            pltpu.sync_copy(z_hbm, acc_sh.at[pl.ds(sid * Bg * rpt, Bg * rpt)])
            plsc.subcore_barrier()

            def batch_body(bl, carry):
                b = cid * (B // NC) + p * Bg + bl

                def off_body(kk, c2):
                    r_ = kk // (jc // 16)
                    k16 = kk % (jc // 16)
                    idxb_v[r_, pl.ds(k16 * 16, 16)] = (
                        idx_v[pl.ds(kk * 16, 16)] + bl * U
                    )
                    return c2

                lax.fori_loop(0, jt // 16, off_body, 0)

                def load_body(q, c2):
                    pltpu.sync_copy(
                        y_hbm.at[b].at[pl.ds(jbase + q * LY, LY)], y_v
                    )

                    def scat_body(r0, c3):
                        r_ = q * spc + r0
                        pltpu.sync_copy(
                            y_v.at[pl.ds(r0 * jc, jc)],
                            acc_sh.at[idxb_v.at[r_]],
                            add=True,
                        )
                        return c3

                    lax.fori_loop(0, spc, scat_body, 0)
                    return c2

                lax.fori_loop(0, nld, load_body, 0)
                return carry

            lax.fori_loop(0, Bg, batch_body, 0)
            plsc.subcore_barrier()

            def out_body(bl, carry):
                b = cid * (B // NC) + p * Bg + bl
                pltpu.sync_copy(
                    acc_sh.at[pl.ds(bl * U + sid * rpt, rpt)],
                    out_hbm.at[b].at[pl.ds(sid * rpt, rpt)],
                )
                return carry

            lax.fori_loop(0, Bg, out_body, 0)
            if p + 1 < passes:
                plsc.subcore_barrier()

    return k


def _matmul_elu_scale(g2, w, bias, dvt):
    M, K = g2.shape
    D = w.shape[1]
    BM = 1024 if M >= 1024 else M

    def body(g_ref, w_ref, b_ref, s_ref, o_ref):
        a = jnp.dot(g_ref[...], w_ref[...], preferred_element_type=jnp.float32)
        a = a + b_ref[...]
        a = jnp.where(a > 0, a, jnp.exp(jnp.minimum(a, 0.0)) - 1.0)
        o_ref[...] = a * s_ref[...]

    return pl.pallas_call(
        body,
        grid=(M // BM,),
        in_specs=[
            pl.BlockSpec((BM, K), lambda r: (r, 0)),
            pl.BlockSpec((K, D), lambda r: (0, 0)),
            pl.BlockSpec((1, D), lambda r: (0, 0)),
            pl.BlockSpec((BM, 1), lambda r: (r, 0)),
        ],
        out_specs=pl.BlockSpec((BM, D), lambda r: (r, 0)),
        out_shape=jax.ShapeDtypeStruct((M, D), jnp.float32),
    )(g2, w, bias, dvt)


def _final_matmul(h, wf, bf):
    M, Kf = h.shape
    D = wf.shape[1]
    BK = 4096

    def body(h_ref, w_ref, b_ref, o_ref):
        @pl.when(pl.program_id(0) == 0)
        def _():
            o_ref[...] = jnp.broadcast_to(b_ref[...], o_ref.shape)

        o_ref[...] += jnp.dot(h_ref[...], w_ref[...], preferred_element_type=jnp.float32)

    return pl.pallas_call(
        body,
        grid=(Kf // BK,),
        in_specs=[
            pl.BlockSpec((M, BK), lambda k: (0, k)),
            pl.BlockSpec((BK, D), lambda k: (k, 0)),
            pl.BlockSpec((1, D), lambda k: (0, 0)),
        ],
        out_specs=pl.BlockSpec((M, D), lambda k: (0, 0)),
        out_shape=jax.ShapeDtypeStruct((M, D), jnp.float32),
    )(h, wf, bf)


def kernel(x, W0, b0, W1, b1, W2, b2, W3, b3, Wf, bf,
           dv0, dv1, dv2, dv3, sp0, sp1, sp2, sp3,
           dr0, dr1, dr2, dr3, dc0, dc1, dc2, dc3):
    Ws = [W0, W1, W2, W3]
    bs = [b0, b1, b2, b3]
    sps = [sp0, sp1, sp2, sp3]
    drs = [dr0, dr1, dr2, dr3]
    dcs = [dc0, dc1, dc2, dc3]
    dvs = [dv0, dv1, dv2, dv3]

    h = jnp.pad(x, ((0, 0), (0, 0), (0, 13)))
    W0p = jnp.zeros((SL, 16, CH[1]), W0.dtype).at[:, :3, :].set(
        W0.reshape(SL, 3, CH[1])
    ).reshape(SL * 16, CH[1])
    Ws = [W0p, W1, W2, W3]
    Cg = [16, 64, 128, 256]

    for i in range(4):
        V, U, C, D = VIN[i], VOUT[i], Cg[i], CH[i + 1]
        N = 4 * U
        jn = N // NW
        dc3 = dcs[i].reshape(NW, jn // _chunk(jn), _chunk(jn))
        idx2 = _make_compose(V, N)(
            jnp.pad(sps[i], ((0, 0), (0, 16 - SL))), dc3
        )
        idxf = idx2[:, :SL].reshape(N * SL)
        R = (N * SL) // NW
        idx3 = idxf.reshape(NW, R // _chunk(R), _chunk(R))
        g = _make_gather(V, C, N)(h, idx3)
        dvt = jnp.tile(dvs[i], B).reshape(B * N, 1)
        y = _matmul_elu_scale(
            g.reshape(B * N, SL * C), Ws[i], bs[i].reshape(1, D), dvt
        )
        Bg = min(B // NC, (6 * 2 ** 20 // 4) // (U * D))
        while (B // NC) % Bg:
            Bg -= 1
        z = jnp.zeros((Bg * (U // NS), D), jnp.float32)
        h = _make_scatter(U, D, N)(
            y.reshape(B, N, D), drs[i].reshape(NS, N // NS), z
        )
    z = _final_matmul(h.reshape(B, VOUT[3] * CH[4]), Wf, bf.reshape(1, LATENT))
    return z


# speedup vs baseline: 1.6210x; 1.0535x over previous
import functools

import jax
import jax.numpy as jnp
from jax import lax
from jax.experimental import pallas as pl
from jax.experimental.pallas import tpu as pltpu
from jax.experimental.pallas import tpu_sc as plsc

B = 16
SL = 9
VIN = [16384, 4096, 1024, 256]
VOUT = [4096, 1024, 256, 64]
CH = [3, 64, 128, 256, 512]
LATENT = 256

NC = 2
NS = 16
NW = NC * NS


def _mesh():
    return plsc.VectorSubcoreMesh(
        core_axis_name="c", subcore_axis_name="s", num_cores=NC, num_subcores=NS
    )


def _wid():
    return lax.axis_index("s") * NC + lax.axis_index("c")


def _chunk(n):
    for c in range(min(n, 128), 0, -1):
        if n % c == 0 and (c % 8 == 0 or c == n):
            return c
    return n


_SC_PARAMS = dict(
    compiler_params=pltpu.CompilerParams(use_tc_tiling_on_sc=False),
)


def _make_compose(V, N):
    jn = N // NW
    CG = _chunk(jn)
    nch = jn // CG

    @functools.partial(
        pl.kernel,
        mesh=_mesh(),
        out_type=jax.ShapeDtypeStruct((N, 16), jnp.int32),
        scratch_types=[
            pltpu.VMEM((nch, CG), jnp.int32),
            pltpu.VMEM((jn, 16), jnp.int32),
            pltpu.SemaphoreType.DMA,
        ],
        **_SC_PARAMS,
    )
    def k(sp_hbm, dc_hbm, out_hbm, dc_v, rows_v, sem):
        base = _wid() * jn
        pltpu.sync_copy(dc_hbm.at[_wid()], dc_v)

        def issue(ci, carry):
            pltpu.async_copy(
                sp_hbm.at[dc_v.at[ci]], rows_v.at[pl.ds(ci * CG, CG)], sem
            )
            return carry

        lax.fori_loop(0, nch, issue, 0)
        pltpu.make_async_copy(
            out_hbm.at[pl.ds(base, jn)], rows_v, sem
        ).wait()
        pltpu.sync_copy(rows_v, out_hbm.at[pl.ds(base, jn)])

    return k


def _make_gather(V, C, N):
    R = (N * SL) // NW
    CG = _chunk(R)
    nch = R // CG
    nbuf = 2 if 2 * R * C * 4 + nch * CG * 4 <= 400_000 else 1

    @functools.partial(
        pl.kernel,
        mesh=_mesh(),
        out_type=jax.ShapeDtypeStruct((B * N * SL, C), jnp.float32),
        scratch_types=[
            pltpu.VMEM((nch, CG), jnp.int32),
            pltpu.VMEM((nbuf, R, C), jnp.float32),
            pltpu.SemaphoreType.DMA,
            pltpu.SemaphoreType.DMA,
            pltpu.SemaphoreType.DMA,
            pltpu.SemaphoreType.DMA,
        ],
        **_SC_PARAMS,
    )
    def k(x_hbm, idx_hbm, g_hbm, idx_v, rows_v, g0, g1, t0, t1):
        wid = _wid()
        base = wid * R
        pltpu.sync_copy(idx_hbm.at[wid], idx_v)
        gsems = [g0, g1]
        ssems = [t0, t1]

        def gdst(b):
            return g_hbm.at[pl.ds(b * N * SL + base, R)]

        def fire(b, buf):
            def issue(ci, c2):
                pltpu.async_copy(
                    x_hbm.at[b].at[idx_v.at[ci]],
                    rows_v.at[buf].at[pl.ds(ci * CG, CG)],
                    gsems[buf],
                )
                return c2

            lax.fori_loop(0, nch, issue, 0)

        def drain_then_store(b, buf):
            pltpu.make_async_copy(gdst(b), rows_v.at[buf], gsems[buf]).wait()
            pltpu.async_copy(rows_v.at[buf], gdst(b), ssems[buf])

        def wait_store(b, buf):
            pltpu.make_async_copy(rows_v.at[buf], gdst(b), ssems[buf]).wait()

        if nbuf == 2:
            fire(0, 0)
            for b in range(1, B):
                buf = b % 2
                if b >= 2:
                    wait_store(b - 2, buf)
                fire(b, buf)
                drain_then_store(b - 1, 1 - buf)
            drain_then_store(B - 1, (B - 1) % 2)
            wait_store(B - 2, B % 2)
            wait_store(B - 1, (B - 1) % 2)
        else:
            def b_body(b, c1):
                fire(b, 0)
                pltpu.make_async_copy(gdst(b), rows_v.at[0], gsems[0]).wait()
                pltpu.sync_copy(rows_v.at[0], gdst(b))
                return c1

            lax.fori_loop(0, B, b_body, 0)

    return k


def _make_scatter(U, D, N):
    jt = N // NS
    bpb = U * D
    Bg = min(B // NC, (6 * 2 ** 20 // 4) // bpb)
    while (B // NC) % Bg:
        Bg -= 1
    passes = (B // NC) // Bg
    rpt = U // NS
    jc = _chunk(jt)
    nchs = jt // jc
    LY = min(jt, 512)
    nld = jt // LY
    spc = LY // jc

    @functools.partial(
        pl.kernel,
        mesh=_mesh(),
        out_type=jax.ShapeDtypeStruct((B, U, D), jnp.float32),
        scratch_types=[
            pltpu.VMEM_SHARED((Bg * U, D), jnp.float32),
            pltpu.VMEM((LY, D), jnp.float32),
            pltpu.VMEM((jt,), jnp.int32),
            pltpu.VMEM((nchs, jc), jnp.int32),
            pltpu.SemaphoreType.DMA,
        ],
        **_SC_PARAMS,
    )
    def k(y_hbm, dr_hbm, z_hbm, out_hbm, acc_sh, y_v, idx_v, idxb_v, sem):
        cid = lax.axis_index("c")
        sid = lax.axis_index("s")
        jbase = sid * jt
        pltpu.sync_copy(dr_hbm.at[sid], idx_v)
        for p in range(passes):
            pltpu.sync_copy(z_hbm, acc_sh.at[pl.ds(sid * Bg * rpt, Bg * rpt)])
            plsc.subcore_barrier()

            def batch_body(bl, carry):
                b = cid * (B // NC) + p * Bg + bl

                def off_body(kk, c2):
                    r_ = kk // (jc // 16)
                    k16 = kk % (jc // 16)
                    idxb_v[r_, pl.ds(k16 * 16, 16)] = (
                        idx_v[pl.ds(kk * 16, 16)] + bl * U
                    )
                    return c2

                lax.fori_loop(0, jt // 16, off_body, 0)

                def load_body(q, c2):
                    pltpu.sync_copy(
                        y_hbm.at[b].at[pl.ds(jbase + q * LY, LY)], y_v
                    )

                    def scat_body(r0, c3):
                        r_ = q * spc + r0
                        pltpu.sync_copy(
                            y_v.at[pl.ds(r0 * jc, jc)],
                            acc_sh.at[idxb_v.at[r_]],
                            add=True,
                        )
                        return c3

                    lax.fori_loop(0, spc, scat_body, 0)
                    return c2

                lax.fori_loop(0, nld, load_body, 0)
                return carry

            lax.fori_loop(0, Bg, batch_body, 0)
            plsc.subcore_barrier()

            def out_body(bl, carry):
                b = cid * (B // NC) + p * Bg + bl
                pltpu.sync_copy(
                    acc_sh.at[pl.ds(bl * U + sid * rpt, rpt)],
                    out_hbm.at[b].at[pl.ds(sid * rpt, rpt)],
                )
                return carry

            lax.fori_loop(0, Bg, out_body, 0)
            if p + 1 < passes:
                plsc.subcore_barrier()

    return k


def _matmul_elu_scale(g2, w, bias, dvt):
    M, K = g2.shape
    D = w.shape[1]
    BM = 2048 if M >= 2048 else M

    def body(g_ref, w_ref, b_ref, s_ref, o_ref):
        a = jnp.dot(g_ref[...], w_ref[...], preferred_element_type=jnp.float32)
        a = a + b_ref[...]
        a = jnp.where(a > 0, a, jnp.exp(jnp.minimum(a, 0.0)) - 1.0)
        o_ref[...] = a * s_ref[...]

    return pl.pallas_call(
        body,
        grid=(M // BM,),
        in_specs=[
            pl.BlockSpec((BM, K), lambda r: (r, 0)),
            pl.BlockSpec((K, D), lambda r: (0, 0)),
            pl.BlockSpec((1, D), lambda r: (0, 0)),
            pl.BlockSpec((BM, 1), lambda r: (r, 0)),
        ],
        out_specs=pl.BlockSpec((BM, D), lambda r: (r, 0)),
        out_shape=jax.ShapeDtypeStruct((M, D), jnp.float32),
    )(g2, w, bias, dvt)


def _final_matmul(h, wf, bf):
    M, Kf = h.shape
    D = wf.shape[1]
    BK = 4096

    def body(h_ref, w_ref, b_ref, o_ref):
        @pl.when(pl.program_id(0) == 0)
        def _():
            o_ref[...] = jnp.broadcast_to(b_ref[...], o_ref.shape)

        o_ref[...] += jnp.dot(h_ref[...], w_ref[...], preferred_element_type=jnp.float32)

    return pl.pallas_call(
        body,
        grid=(Kf // BK,),
        in_specs=[
            pl.BlockSpec((M, BK), lambda k: (0, k)),
            pl.BlockSpec((BK, D), lambda k: (k, 0)),
            pl.BlockSpec((1, D), lambda k: (0, 0)),
        ],
        out_specs=pl.BlockSpec((M, D), lambda k: (0, 0)),
        out_shape=jax.ShapeDtypeStruct((M, D), jnp.float32),
    )(h, wf, bf)


def kernel(x, W0, b0, W1, b1, W2, b2, W3, b3, Wf, bf,
           dv0, dv1, dv2, dv3, sp0, sp1, sp2, sp3,
           dr0, dr1, dr2, dr3, dc0, dc1, dc2, dc3):
    Ws = [W0, W1, W2, W3]
    bs = [b0, b1, b2, b3]
    sps = [sp0, sp1, sp2, sp3]
    drs = [dr0, dr1, dr2, dr3]
    dcs = [dc0, dc1, dc2, dc3]
    dvs = [dv0, dv1, dv2, dv3]

    h = jnp.pad(x, ((0, 0), (0, 0), (0, 13)))
    W0p = jnp.zeros((SL, 16, CH[1]), W0.dtype).at[:, :3, :].set(
        W0.reshape(SL, 3, CH[1])
    ).reshape(SL * 16, CH[1])
    Ws = [W0p, W1, W2, W3]
    Cg = [16, 64, 128, 256]

    for i in range(4):
        V, U, C, D = VIN[i], VOUT[i], Cg[i], CH[i + 1]
        N = 4 * U
        jn = N // NW
        dc3 = dcs[i].reshape(NW, jn // _chunk(jn), _chunk(jn))
        idx2 = _make_compose(V, N)(
            jnp.pad(sps[i], ((0, 0), (0, 16 - SL))), dc3
        )
        idxf = idx2[:, :SL].reshape(N * SL)
        R = (N * SL) // NW
        idx3 = idxf.reshape(NW, R // _chunk(R), _chunk(R))
        g = _make_gather(V, C, N)(h, idx3)
        dvt = jnp.tile(dvs[i], B).reshape(B * N, 1)
        y = _matmul_elu_scale(
            g.reshape(B * N, SL * C), Ws[i], bs[i].reshape(1, D), dvt
        )
        Bg = min(B // NC, (6 * 2 ** 20 // 4) // (U * D))
        while (B // NC) % Bg:
            Bg -= 1
        z = jnp.zeros((Bg * (U // NS), D), jnp.float32)
        h = _make_scatter(U, D, N)(
            y.reshape(B, N, D), drs[i].reshape(NS, N // NS), z
        )
    z = _final_matmul(h.reshape(B, VOUT[3] * CH[4]), Wf, bf.reshape(1, LATENT))
    return z
